# baseline (speedup 1.0000x reference)
import functools

import jax
import jax.numpy as jnp
from jax import lax
from jax.experimental import pallas as pl
from jax.experimental.pallas import tpu as pltpu
from jax.experimental.pallas import tpu_sc as plsc

N = 10000
E = 320000
H = 128
EMB = 16

NC = 2
NS = 16
NW = NC * NS
EW = E // NW
C = 80
NCH = EW // C
RPS = 632
NPAD = RPS * NS

_TE = 4000


def _pack_bf16_pair(lo, hi):
    lo16 = jax.lax.bitcast_convert_type(
        lo.astype(jnp.bfloat16), jnp.uint16).astype(jnp.uint32)
    hi16 = jax.lax.bitcast_convert_type(
        hi.astype(jnp.bfloat16), jnp.uint16).astype(jnp.uint32)
    return jax.lax.bitcast_convert_type(lo16 | (hi16 << 16), jnp.int32)


def _gates_body(ea_ref, pv_ref, wee_ref, weo_ref, wc0e_ref, wc0o_ref,
                wc1e_ref, wc1o_ref, g1_ref, g2_ref):
    ea = ea_ref[...]
    pv = pv_ref[...]
    m = (pv[:, 0:1] != 0.0).astype(jnp.float32)

    def halves(we_r, wc0_r, wc1_r):
        ec = jnp.maximum(
            jnp.dot(ea, we_r[...], preferred_element_type=jnp.float32), 0.0)
        c1 = ec + jnp.dot(pv, wc0_r[...], preferred_element_type=jnp.float32) * m
        c2 = c1 + jnp.dot(pv, wc1_r[...], preferred_element_type=jnp.float32)
        return 1.0 / (1.0 + jnp.exp(-c1)), 1.0 / (1.0 + jnp.exp(-c2))

    g1e, g2e = halves(wee_ref, wc0e_ref, wc1e_ref)
    g1o, g2o = halves(weo_ref, wc0o_ref, wc1o_ref)
    g1_ref[...] = _pack_bf16_pair(g1e, g1o)
    g2_ref[...] = _pack_bf16_pair(g2e, g2o)


def _gates(ea, pv, wee, weo, wc0e, wc0o, wc1e, wc1o):
    grid = (E // _TE,)
    wspec = pl.BlockSpec((EMB, H // 2), lambda i: (0, 0))
    gspec = pl.BlockSpec((_TE, H // 2), lambda i: (i, 0))
    return pl.pallas_call(
        _gates_body,
        grid=grid,
        in_specs=[
            pl.BlockSpec((_TE, EMB), lambda i: (i, 0)),
            pl.BlockSpec((_TE, EMB), lambda i: (i, 0)),
            wspec, wspec, wspec, wspec, wspec, wspec,
        ],
        out_specs=[gspec, gspec],
        out_shape=[
            jax.ShapeDtypeStruct((E, H // 2), jnp.int32),
            jax.ShapeDtypeStruct((E, H // 2), jnp.int32),
        ],
        compiler_params=pltpu.CompilerParams(
            dimension_semantics=(pltpu.PARALLEL,)),
    )(ea, pv, wee, weo, wc0e, wc0o, wc1e, wc1o)


_TN = 2000


def _node_in_body(x_ref, lv_ref, wn_ref, b_ref, wl_ref, wm_ref,
                  hpre_ref, hm_ref):
    h = jnp.maximum(
        jnp.dot(x_ref[...], wn_ref[...], preferred_element_type=jnp.float32)
        + b_ref[...], 0.0)
    hp = h + jnp.dot(lv_ref[...], wl_ref[...], preferred_element_type=jnp.float32)
    hpre_ref[...] = hp
    hm_ref[...] = jnp.dot(hp, wm_ref[...], preferred_element_type=jnp.float32)


def _node_in(x, lv, wn, b, wl, wm):
    grid = (N // _TN,)
    return pl.pallas_call(
        _node_in_body,
        grid=grid,
        in_specs=[
            pl.BlockSpec((_TN, x.shape[1]), lambda i: (i, 0)),
            pl.BlockSpec((_TN, EMB), lambda i: (i, 0)),
            pl.BlockSpec((x.shape[1], H), lambda i: (0, 0)),
            pl.BlockSpec((1, H), lambda i: (0, 0)),
            pl.BlockSpec((EMB, H), lambda i: (0, 0)),
            pl.BlockSpec((H, H), lambda i: (0, 0)),
        ],
        out_specs=[
            pl.BlockSpec((_TN, H), lambda i: (i, 0)),
            pl.BlockSpec((_TN, H), lambda i: (i, 0)),
        ],
        out_shape=[
            jax.ShapeDtypeStruct((N, H), jnp.float32),
            jax.ShapeDtypeStruct((N, H), jnp.float32),
        ],
        compiler_params=pltpu.CompilerParams(
            dimension_semantics=(pltpu.PARALLEL,)),
    )(x, lv, wn, b, wl, wm)


def _node_mid_body(hpre_ref, p0_ref, p1_ref, lv_ref, wu_ref, wl_ref,
                   wm_ref, hpre1_ref, hm1_ref):
    agg = p0_ref[...] + p1_ref[...]
    h1 = jnp.maximum(
        hpre_ref[...]
        + jnp.dot(agg, wu_ref[...], preferred_element_type=jnp.float32), 0.0)
    hp1 = h1 + jnp.dot(lv_ref[...], wl_ref[...], preferred_element_type=jnp.float32)
    hpre1_ref[...] = hp1
    hm1_ref[...] = jnp.dot(hp1, wm_ref[...], preferred_element_type=jnp.float32)


def _node_mid(hpre, p0, p1, lv, wu, wl, wm):
    grid = (N // _TN,)
    return pl.pallas_call(
        _node_mid_body,
        grid=grid,
        in_specs=[
            pl.BlockSpec((_TN, H), lambda i: (i, 0)),
            pl.BlockSpec((_TN, H), lambda i: (i, 0)),
            pl.BlockSpec((_TN, H), lambda i: (i, 0)),
            pl.BlockSpec((_TN, EMB), lambda i: (i, 0)),
            pl.BlockSpec((H, H), lambda i: (0, 0)),
            pl.BlockSpec((EMB, H), lambda i: (0, 0)),
            pl.BlockSpec((H, H), lambda i: (0, 0)),
        ],
        out_specs=[
            pl.BlockSpec((_TN, H), lambda i: (i, 0)),
            pl.BlockSpec((_TN, H), lambda i: (i, 0)),
        ],
        out_shape=[
            jax.ShapeDtypeStruct((N, H), jnp.float32),
            jax.ShapeDtypeStruct((N, H), jnp.float32),
        ],
        compiler_params=pltpu.CompilerParams(
            dimension_semantics=(pltpu.PARALLEL,)),
    )(hpre, p0, p1, lv, wu, wl, wm)


def _node_out_body(hpre_ref, p0_ref, p1_ref, wu_ref, wh_ref, out_ref):
    agg = p0_ref[...] + p1_ref[...]
    h2 = jnp.maximum(
        hpre_ref[...]
        + jnp.dot(agg, wu_ref[...], preferred_element_type=jnp.float32), 0.0)
    out_ref[...] = jnp.dot(h2, wh_ref[...], preferred_element_type=jnp.float32)


def _node_out(hpre, p0, p1, wu, wh):
    grid = (N // _TN,)
    return pl.pallas_call(
        _node_out_body,
        grid=grid,
        in_specs=[
            pl.BlockSpec((_TN, H), lambda i: (i, 0)),
            pl.BlockSpec((_TN, H), lambda i: (i, 0)),
            pl.BlockSpec((_TN, H), lambda i: (i, 0)),
            pl.BlockSpec((H, H), lambda i: (0, 0)),
            pl.BlockSpec((H, wh.shape[1]), lambda i: (0, 0)),
        ],
        out_specs=pl.BlockSpec((_TN, wh.shape[1]), lambda i: (i, 0)),
        out_shape=jax.ShapeDtypeStruct((N, wh.shape[1]), jnp.float32),
        compiler_params=pltpu.CompilerParams(
            dimension_semantics=(pltpu.PARALLEL,)),
    )(hpre, p0, p1, wu, wh)


_sc_mesh = plsc.VectorSubcoreMesh(
    core_axis_name="c", subcore_axis_name="s", num_cores=NC, num_subcores=NS)


@functools.partial(
    pl.kernel,
    out_type=jax.ShapeDtypeStruct((NC, NPAD, H), jnp.float32),
    mesh=_sc_mesh,
    compiler_params=pltpu.CompilerParams(needs_layout_passes=False),
    scratch_types=[
        pltpu.VMEM((2 * C,), jnp.int32),
        pltpu.VMEM((2 * C,), jnp.int32),
        pltpu.VMEM((C,), jnp.int32),
        pltpu.VMEM((C, H), jnp.float32),
        pltpu.VMEM((C, H), jnp.float32),
        pltpu.VMEM((C, H // 2), jnp.int32),
        pltpu.VMEM((C, H // 2), jnp.int32),
        pltpu.VMEM_SHARED((NPAD, H), jnp.float32),
        pltpu.SemaphoreType.DMA,
        pltpu.SemaphoreType.DMA,
        pltpu.SemaphoreType.DMA,
        pltpu.SemaphoreType.DMA,
        pltpu.SemaphoreType.DMA,
        pltpu.SemaphoreType.DMA,
    ],
)
def _sc_edge(hm_hbm, g_hbm, idx_hbm, zero_hbm, out_hbm,
             ibuf0_v, ibuf1_v, dchunk_v,
             rows0_v, rows1_v, gate0_v, gate1_v,
             acc_sh, semr0, semr1, seml0, seml1, semi0, semi1):
    c = lax.axis_index("c")
    s = lax.axis_index("s")
    wid = s * NC + c
    ibuf = (ibuf0_v, ibuf1_v)
    rows_v = (rows0_v, rows1_v)
    gate_v = (gate0_v, gate1_v)
    semr = (semr0, semr1)
    seml = (seml0, seml1)
    semi = (semi0, semi1)

    pltpu.sync_copy(zero_hbm.at[pl.ds(s * RPS, RPS)],
                    acc_sh.at[pl.ds(s * RPS, RPS)])
    plsc.subcore_barrier()

    def issue_idx(i, b):
        pltpu.async_copy(idx_hbm.at[pl.ds((wid * NCH + i) * 2 * C, 2 * C)],
                         ibuf[b], semi[b])

    def wait_idx(b):
        pltpu.make_async_copy(idx_hbm.at[pl.ds(0, 2 * C)], ibuf[b],
                              semi[b]).wait()

    def issue_data(i, b):
        pltpu.async_copy(hm_hbm.at[ibuf[b].at[pl.ds(0, C)]],
                         rows_v[b], semr[b])
        pltpu.async_copy(g_hbm.at[pl.ds(wid * EW + i * C, C)],
                         gate_v[b], seml[b])

    def drain_data(b):
        pltpu.make_async_copy(hm_hbm.at[pl.ds(0, C)], rows_v[b], semr[b]).wait()
        pltpu.make_async_copy(g_hbm.at[pl.ds(0, C)], gate_v[b], seml[b]).wait()

    def gate_and_scatter(b):
        @plsc.parallel_loop(0, C, 1, unroll=4)
        def mul_row(r):
            for k in range(H // 32):
                gbf = plsc.bitcast(gate_v[b][r, pl.ds(k * 16, 16)],
                                   jnp.bfloat16)
                ga, gb = plsc.unpack(gbf, format=plsc.PackFormat.INTERLEAVED)
                sla = pl.ds(k * 32, 16)
                slb = pl.ds(k * 32 + 16, 16)
                rows_v[b][r, sla] = rows_v[b][r, sla] * ga
                rows_v[b][r, slb] = rows_v[b][r, slb] * gb
        for k in range(C // 16):
            dchunk_v[pl.ds(k * 16, 16)] = ibuf[b][pl.ds(C + k * 16, 16)]
        pltpu.sync_copy(rows_v[b], acc_sh.at[dchunk_v], add=True)

    issue_idx(0, 0)
    wait_idx(0)
    issue_data(0, 0)
    issue_idx(1, 1)

    def step(st, carry):
        for b in range(2):
            i = 2 * st + b
            wait_idx(1 - b)
            issue_data(i + 1, 1 - b)
            drain_data(b)
            gate_and_scatter(b)

            @pl.when(i + 2 < NCH)
            def _():
                issue_idx(i + 2, b)
        return carry

    lax.fori_loop(0, (NCH - 1) // 2, step, 0)
    drain_data(0)
    gate_and_scatter(0)
    plsc.subcore_barrier()

    pltpu.sync_copy(acc_sh.at[pl.ds(s * RPS, RPS)],
                    out_hbm.at[c, pl.ds(s * RPS, RPS)])


import numpy as _np

_PEV = _np.empty(H // 2, dtype=_np.int32)
_POD = _np.empty(H // 2, dtype=_np.int32)
for _k in range(H // 32):
    for _j in range(16):
        _PEV[16 * _k + _j] = 32 * _k + _j
        _POD[16 * _k + _j] = 32 * _k + 16 + _j


def kernel(x, edge_index, edge_attr, loop_val, poly_val,
           W_node, b_node, W_edge_enc, W_loop, W_conn, W_msg, W_upd, W_head):
    src = edge_index[0].astype(jnp.int32)
    dst = edge_index[1].astype(jnp.int32)
    wee, weo = W_edge_enc[:, _PEV], W_edge_enc[:, _POD]
    wc0e, wc0o = W_conn[0][:, _PEV], W_conn[0][:, _POD]
    wc1e, wc1o = W_conn[1][:, _PEV], W_conn[1][:, _POD]
    idx2 = jnp.stack([src.reshape(-1, C), dst.reshape(-1, C)],
                     axis=1).reshape(-1)
    zeros = jnp.zeros((NPAD, H), jnp.float32)

    g1, g2 = _gates(edge_attr, poly_val, wee, weo, wc0e, wc0o, wc1e, wc1o)

    hpre0, hm0 = _node_in(x, loop_val, W_node, b_node.reshape(1, H),
                          W_loop[0], W_msg[0])
    part0 = _sc_edge(hm0, g1, idx2, zeros)
    hpre1, hm1 = _node_mid(hpre0, part0[0, :N], part0[1, :N], loop_val,
                           W_upd[0], W_loop[1], W_msg[1])
    part1 = _sc_edge(hm1, g2, idx2, zeros)
    out = _node_out(hpre1, part1[0, :N], part1[1, :N], W_upd[1], W_head)
    return out


# speedup vs baseline: 1.0544x; 1.0544x over previous
import functools

import jax
import jax.numpy as jnp
from jax import lax
from jax.experimental import pallas as pl
from jax.experimental.pallas import tpu as pltpu
from jax.experimental.pallas import tpu_sc as plsc

N = 10000
E = 320000
H = 128
EMB = 16

NC = 2
NS = 16
NW = NC * NS
EW = E // NW
C = 80
NCH = EW // C
RPS = 632
NPAD = RPS * NS

_TE = 4000


def _pack_bf16_pair(lo, hi):
    lo16 = jax.lax.bitcast_convert_type(
        lo.astype(jnp.bfloat16), jnp.uint16).astype(jnp.uint32)
    hi16 = jax.lax.bitcast_convert_type(
        hi.astype(jnp.bfloat16), jnp.uint16).astype(jnp.uint32)
    return jax.lax.bitcast_convert_type(lo16 | (hi16 << 16), jnp.int32)


def _gates1_body(ea_ref, pv_ref, wee_ref, weo_ref, wc0e_ref, wc0o_ref, g1_ref):
    ea = ea_ref[...]
    pv = pv_ref[...]
    m = (pv[:, 0:1] != 0.0).astype(jnp.float32)

    def half(we_r, wc_r):
        ec = jnp.maximum(
            jnp.dot(ea, we_r[...], preferred_element_type=jnp.float32), 0.0)
        c1 = ec + jnp.dot(pv, wc_r[...], preferred_element_type=jnp.float32) * m
        return 1.0 / (1.0 + jnp.exp(-c1))

    g1_ref[...] = _pack_bf16_pair(half(wee_ref, wc0e_ref),
                                  half(weo_ref, wc0o_ref))


def _gates2_body(ea_ref, pv_ref, wee_ref, weo_ref, wc0e_ref, wc0o_ref,
                 wc1e_ref, wc1o_ref, g2_ref):
    ea = ea_ref[...]
    pv = pv_ref[...]
    m = (pv[:, 0:1] != 0.0).astype(jnp.float32)

    def half(we_r, wc0_r, wc1_r):
        ec = jnp.maximum(
            jnp.dot(ea, we_r[...], preferred_element_type=jnp.float32), 0.0)
        c1 = ec + jnp.dot(pv, wc0_r[...], preferred_element_type=jnp.float32) * m
        c2 = c1 + jnp.dot(pv, wc1_r[...], preferred_element_type=jnp.float32)
        return 1.0 / (1.0 + jnp.exp(-c2))

    g2_ref[...] = _pack_bf16_pair(half(wee_ref, wc0e_ref, wc1e_ref),
                                  half(weo_ref, wc0o_ref, wc1o_ref))


def _gates1(ea, pv, wee, weo, wc0e, wc0o):
    grid = (E // _TE,)
    wspec = pl.BlockSpec((EMB, H // 2), lambda i: (0, 0))
    return pl.pallas_call(
        _gates1_body,
        grid=grid,
        in_specs=[
            pl.BlockSpec((_TE, EMB), lambda i: (i, 0)),
            pl.BlockSpec((_TE, EMB), lambda i: (i, 0)),
            wspec, wspec, wspec, wspec,
        ],
        out_specs=pl.BlockSpec((_TE, H // 2), lambda i: (i, 0)),
        out_shape=jax.ShapeDtypeStruct((E, H // 2), jnp.int32),
        compiler_params=pltpu.CompilerParams(
            dimension_semantics=(pltpu.PARALLEL,)),
    )(ea, pv, wee, weo, wc0e, wc0o)


def _gates2(ea, pv, wee, weo, wc0e, wc0o, wc1e, wc1o):
    grid = (E // _TE,)
    wspec = pl.BlockSpec((EMB, H // 2), lambda i: (0, 0))
    return pl.pallas_call(
        _gates2_body,
        grid=grid,
        in_specs=[
            pl.BlockSpec((_TE, EMB), lambda i: (i, 0)),
            pl.BlockSpec((_TE, EMB), lambda i: (i, 0)),
            wspec, wspec, wspec, wspec, wspec, wspec,
        ],
        out_specs=pl.BlockSpec((_TE, H // 2), lambda i: (i, 0)),
        out_shape=jax.ShapeDtypeStruct((E, H // 2), jnp.int32),
        compiler_params=pltpu.CompilerParams(
            dimension_semantics=(pltpu.PARALLEL,)),
    )(ea, pv, wee, weo, wc0e, wc0o, wc1e, wc1o)


_TN = 2000


def _node_in_body(x_ref, lv_ref, wn_ref, b_ref, wl_ref, wm_ref,
                  hpre_ref, hm_ref):
    h = jnp.maximum(
        jnp.dot(x_ref[...], wn_ref[...], preferred_element_type=jnp.float32)
        + b_ref[...], 0.0)
    hp = h + jnp.dot(lv_ref[...], wl_ref[...], preferred_element_type=jnp.float32)
    hpre_ref[...] = hp
    hm_ref[...] = jnp.dot(hp, wm_ref[...], preferred_element_type=jnp.float32)


def _node_in(x, lv, wn, b, wl, wm):
    grid = (N // _TN,)
    return pl.pallas_call(
        _node_in_body,
        grid=grid,
        in_specs=[
            pl.BlockSpec((_TN, x.shape[1]), lambda i: (i, 0)),
            pl.BlockSpec((_TN, EMB), lambda i: (i, 0)),
            pl.BlockSpec((x.shape[1], H), lambda i: (0, 0)),
            pl.BlockSpec((1, H), lambda i: (0, 0)),
            pl.BlockSpec((EMB, H), lambda i: (0, 0)),
            pl.BlockSpec((H, H), lambda i: (0, 0)),
        ],
        out_specs=[
            pl.BlockSpec((_TN, H), lambda i: (i, 0)),
            pl.BlockSpec((_TN, H), lambda i: (i, 0)),
        ],
        out_shape=[
            jax.ShapeDtypeStruct((N, H), jnp.float32),
            jax.ShapeDtypeStruct((N, H), jnp.float32),
        ],
        compiler_params=pltpu.CompilerParams(
            dimension_semantics=(pltpu.PARALLEL,)),
    )(x, lv, wn, b, wl, wm)


def _node_mid_body(hpre_ref, part_ref, lv_ref, wu_ref, wl_ref,
                   wm_ref, hpre1_ref, hm1_ref):
    agg = part_ref[0] + part_ref[1]
    h1 = jnp.maximum(
        hpre_ref[...]
        + jnp.dot(agg, wu_ref[...], preferred_element_type=jnp.float32), 0.0)
    hp1 = h1 + jnp.dot(lv_ref[...], wl_ref[...], preferred_element_type=jnp.float32)
    hpre1_ref[...] = hp1
    hm1_ref[...] = jnp.dot(hp1, wm_ref[...], preferred_element_type=jnp.float32)


def _node_mid(hpre, part, lv, wu, wl, wm):
    grid = (N // _TN,)
    return pl.pallas_call(
        _node_mid_body,
        grid=grid,
        in_specs=[
            pl.BlockSpec((_TN, H), lambda i: (i, 0)),
            pl.BlockSpec((NC, _TN, H), lambda i: (0, i, 0)),
            pl.BlockSpec((_TN, EMB), lambda i: (i, 0)),
            pl.BlockSpec((H, H), lambda i: (0, 0)),
            pl.BlockSpec((EMB, H), lambda i: (0, 0)),
            pl.BlockSpec((H, H), lambda i: (0, 0)),
        ],
        out_specs=[
            pl.BlockSpec((_TN, H), lambda i: (i, 0)),
            pl.BlockSpec((_TN, H), lambda i: (i, 0)),
        ],
        out_shape=[
            jax.ShapeDtypeStruct((N, H), jnp.float32),
            jax.ShapeDtypeStruct((N, H), jnp.float32),
        ],
        compiler_params=pltpu.CompilerParams(
            dimension_semantics=(pltpu.PARALLEL,)),
    )(hpre, part, lv, wu, wl, wm)


def _node_out_body(hpre_ref, part_ref, wu_ref, wh_ref, out_ref):
    agg = part_ref[0] + part_ref[1]
    h2 = jnp.maximum(
        hpre_ref[...]
        + jnp.dot(agg, wu_ref[...], preferred_element_type=jnp.float32), 0.0)
    out_ref[...] = jnp.dot(h2, wh_ref[...], preferred_element_type=jnp.float32)


def _node_out(hpre, part, wu, wh):
    grid = (N // _TN,)
    return pl.pallas_call(
        _node_out_body,
        grid=grid,
        in_specs=[
            pl.BlockSpec((_TN, H), lambda i: (i, 0)),
            pl.BlockSpec((NC, _TN, H), lambda i: (0, i, 0)),
            pl.BlockSpec((H, H), lambda i: (0, 0)),
            pl.BlockSpec((H, wh.shape[1]), lambda i: (0, 0)),
        ],
        out_specs=pl.BlockSpec((_TN, wh.shape[1]), lambda i: (i, 0)),
        out_shape=jax.ShapeDtypeStruct((N, wh.shape[1]), jnp.float32),
        compiler_params=pltpu.CompilerParams(
            dimension_semantics=(pltpu.PARALLEL,)),
    )(hpre, part, wu, wh)


_sc_mesh = plsc.VectorSubcoreMesh(
    core_axis_name="c", subcore_axis_name="s", num_cores=NC, num_subcores=NS)


@functools.partial(
    pl.kernel,
    out_type=jax.ShapeDtypeStruct((NC, NPAD, H), jnp.float32),
    mesh=_sc_mesh,
    compiler_params=pltpu.CompilerParams(needs_layout_passes=False),
    scratch_types=[
        pltpu.VMEM((2 * C,), jnp.int32),
        pltpu.VMEM((2 * C,), jnp.int32),
        pltpu.VMEM((C,), jnp.int32),
        pltpu.VMEM((C,), jnp.int32),
        pltpu.VMEM((C, H), jnp.float32),
        pltpu.VMEM((C, H), jnp.float32),
        pltpu.VMEM((C, H // 2), jnp.int32),
        pltpu.VMEM((C, H // 2), jnp.int32),
        pltpu.VMEM_SHARED((NPAD, H), jnp.float32),
        pltpu.SemaphoreType.DMA,
        pltpu.SemaphoreType.DMA,
        pltpu.SemaphoreType.DMA,
        pltpu.SemaphoreType.DMA,
        pltpu.SemaphoreType.DMA,
        pltpu.SemaphoreType.DMA,
        pltpu.SemaphoreType.DMA,
        pltpu.SemaphoreType.DMA,
    ],
)
def _sc_edge(hm_hbm, g_hbm, idx_hbm, zero_hbm, out_hbm,
             ibuf0_v, ibuf1_v, dchunk0_v, dchunk1_v,
             rows0_v, rows1_v, gate0_v, gate1_v,
             acc_sh, semr0, semr1, seml0, seml1, semi0, semi1, sems0, sems1):
    c = lax.axis_index("c")
    s = lax.axis_index("s")
    wid = s * NC + c
    ibuf = (ibuf0_v, ibuf1_v)
    dchunk_v = (dchunk0_v, dchunk1_v)
    rows_v = (rows0_v, rows1_v)
    gate_v = (gate0_v, gate1_v)
    semr = (semr0, semr1)
    seml = (seml0, seml1)
    semi = (semi0, semi1)
    sems = (sems0, sems1)

    pltpu.sync_copy(zero_hbm.at[pl.ds(s * RPS, RPS)],
                    acc_sh.at[pl.ds(s * RPS, RPS)])
    plsc.subcore_barrier()

    def issue_idx(i, b):
        pltpu.async_copy(idx_hbm.at[pl.ds((wid * NCH + i) * 2 * C, 2 * C)],
                         ibuf[b], semi[b])

    def wait_idx(b):
        pltpu.make_async_copy(idx_hbm.at[pl.ds(0, 2 * C)], ibuf[b],
                              semi[b]).wait()

    def issue_data(i, b):
        @pl.when(i >= 2)
        def _():
            pltpu.make_async_copy(hm_hbm.at[pl.ds(0, C)], rows_v[b],
                                  sems[b]).wait()
        pltpu.async_copy(hm_hbm.at[ibuf[b].at[pl.ds(0, C)]],
                         rows_v[b], semr[b])
        pltpu.async_copy(g_hbm.at[pl.ds(wid * EW + i * C, C)],
                         gate_v[b], seml[b])

    def drain_data(b):
        pltpu.make_async_copy(hm_hbm.at[pl.ds(0, C)], rows_v[b], semr[b]).wait()
        pltpu.make_async_copy(g_hbm.at[pl.ds(0, C)], gate_v[b], seml[b]).wait()

    def gate_and_scatter(b):
        @plsc.parallel_loop(0, C, 1, unroll=4)
        def mul_row(r):
            for k in range(H // 32):
                gbf = plsc.bitcast(gate_v[b][r, pl.ds(k * 16, 16)],
                                   jnp.bfloat16)
                ga, gb = plsc.unpack(gbf, format=plsc.PackFormat.INTERLEAVED)
                sla = pl.ds(k * 32, 16)
                slb = pl.ds(k * 32 + 16, 16)
                rows_v[b][r, sla] = rows_v[b][r, sla] * ga
                rows_v[b][r, slb] = rows_v[b][r, slb] * gb
        for k in range(C // 16):
            dchunk_v[b][pl.ds(k * 16, 16)] = ibuf[b][pl.ds(C + k * 16, 16)]
        pltpu.async_copy(rows_v[b], acc_sh.at[dchunk_v[b]], sems[b], add=True)

    issue_idx(0, 0)
    wait_idx(0)
    issue_data(0, 0)
    issue_idx(1, 1)

    def step(st, carry):
        for b in range(2):
            i = 2 * st + b
            wait_idx(1 - b)
            issue_data(i + 1, 1 - b)
            drain_data(b)
            gate_and_scatter(b)

            @pl.when(i + 2 < NCH)
            def _():
                issue_idx(i + 2, b)
        return carry

    lax.fori_loop(0, (NCH - 1) // 2, step, 0)
    drain_data(0)
    gate_and_scatter(0)
    pltpu.make_async_copy(hm_hbm.at[pl.ds(0, C)], rows_v[1], sems[1]).wait()
    pltpu.make_async_copy(hm_hbm.at[pl.ds(0, C)], rows_v[0], sems[0]).wait()
    plsc.subcore_barrier()

    pltpu.sync_copy(acc_sh.at[pl.ds(s * RPS, RPS)],
                    out_hbm.at[c, pl.ds(s * RPS, RPS)])


import numpy as _np

_PEV = _np.empty(H // 2, dtype=_np.int32)
_POD = _np.empty(H // 2, dtype=_np.int32)
for _k in range(H // 32):
    for _j in range(16):
        _PEV[16 * _k + _j] = 32 * _k + _j
        _POD[16 * _k + _j] = 32 * _k + 16 + _j


def kernel(x, edge_index, edge_attr, loop_val, poly_val,
           W_node, b_node, W_edge_enc, W_loop, W_conn, W_msg, W_upd, W_head):
    src = edge_index[0].astype(jnp.int32)
    dst = edge_index[1].astype(jnp.int32)
    wee, weo = W_edge_enc[:, _PEV], W_edge_enc[:, _POD]
    wc0e, wc0o = W_conn[0][:, _PEV], W_conn[0][:, _POD]
    wc1e, wc1o = W_conn[1][:, _PEV], W_conn[1][:, _POD]
    idx2 = jnp.stack([src.reshape(-1, C), dst.reshape(-1, C)],
                     axis=1).reshape(-1)
    zeros = jnp.zeros((NPAD, H), jnp.float32)

    g1 = _gates1(edge_attr, poly_val, wee, weo, wc0e, wc0o)

    hpre0, hm0 = _node_in(x, loop_val, W_node, b_node.reshape(1, H),
                          W_loop[0], W_msg[0])
    part0 = _sc_edge(hm0, g1, idx2, zeros)
    g2 = _gates2(edge_attr, poly_val, wee, weo, wc0e, wc0o, wc1e, wc1o)
    hpre1, hm1 = _node_mid(hpre0, part0, loop_val,
                           W_upd[0], W_loop[1], W_msg[1])
    part1 = _sc_edge(hm1, g2, idx2, zeros)
    out = _node_out(hpre1, part1, W_upd[1], W_head)
    return out


# speedup vs baseline: 1.0955x; 1.0390x over previous
import functools

import jax
import jax.numpy as jnp
import numpy as np
from jax import lax
from jax.experimental import pallas as pl
from jax.experimental.pallas import tpu as pltpu
from jax.experimental.pallas import tpu_sc as plsc

N = 10000
E = 320000
H = 128
EMB = 16

NC = 2
NS = 16
NW = NC * NS
C = 80
RPS = 632
NPAD = RPS * NS

EA = 192000
EB = E - EA

_TE = 4000


def _pack_bf16_pair(lo, hi):
    lo16 = jax.lax.bitcast_convert_type(
        lo.astype(jnp.bfloat16), jnp.uint16).astype(jnp.uint32)
    hi16 = jax.lax.bitcast_convert_type(
        hi.astype(jnp.bfloat16), jnp.uint16).astype(jnp.uint32)
    return jax.lax.bitcast_convert_type(lo16 | (hi16 << 16), jnp.int32)


def _gates_body(ea_ref, pv_ref, wee_ref, weo_ref, wc0e_ref, wc0o_ref,
                wc1e_ref, wc1o_ref, g1_ref, g2_ref):
    ea = ea_ref[...]
    pv = pv_ref[...]
    m = (pv[:, 0:1] != 0.0).astype(jnp.float32)

    def halves(we_r, wc0_r, wc1_r):
        ec = jnp.maximum(
            jnp.dot(ea, we_r[...], preferred_element_type=jnp.float32), 0.0)
        c1 = ec + jnp.dot(pv, wc0_r[...], preferred_element_type=jnp.float32) * m
        c2 = c1 + jnp.dot(pv, wc1_r[...], preferred_element_type=jnp.float32)
        return 1.0 / (1.0 + jnp.exp(-c1)), 1.0 / (1.0 + jnp.exp(-c2))

    g1e, g2e = halves(wee_ref, wc0e_ref, wc1e_ref)
    g1o, g2o = halves(weo_ref, wc0o_ref, wc1o_ref)
    g1_ref[...] = _pack_bf16_pair(g1e, g1o)
    g2_ref[...] = _pack_bf16_pair(g2e, g2o)


def _gates_rng(ea, pv, ws, off_tiles, ne):
    grid = (ne // _TE,)
    wspec = pl.BlockSpec((EMB, H // 2), lambda i: (0, 0))
    gspec = pl.BlockSpec((_TE, H // 2), lambda i: (i, 0))
    return pl.pallas_call(
        _gates_body,
        grid=grid,
        in_specs=[
            pl.BlockSpec((_TE, EMB), lambda i: (i + off_tiles, 0)),
            pl.BlockSpec((_TE, EMB), lambda i: (i + off_tiles, 0)),
            wspec, wspec, wspec, wspec, wspec, wspec,
        ],
        out_specs=[gspec, gspec],
        out_shape=[
            jax.ShapeDtypeStruct((ne, H // 2), jnp.int32),
            jax.ShapeDtypeStruct((ne, H // 2), jnp.int32),
        ],
        compiler_params=pltpu.CompilerParams(
            dimension_semantics=(pltpu.PARALLEL,)),
    )(ea, pv, *ws)


_TN = 2000


def _node_in_body(x_ref, lv_ref, wn_ref, b_ref, wl_ref, wm_ref,
                  hpre_ref, hm_ref):
    h = jnp.maximum(
        jnp.dot(x_ref[...], wn_ref[...], preferred_element_type=jnp.float32)
        + b_ref[...], 0.0)
    hp = h + jnp.dot(lv_ref[...], wl_ref[...], preferred_element_type=jnp.float32)
    hpre_ref[...] = hp
    hm_ref[...] = jnp.dot(hp, wm_ref[...], preferred_element_type=jnp.float32)


def _node_in(x, lv, wn, b, wl, wm):
    grid = (N // _TN,)
    return pl.pallas_call(
        _node_in_body,
        grid=grid,
        in_specs=[
            pl.BlockSpec((_TN, x.shape[1]), lambda i: (i, 0)),
            pl.BlockSpec((_TN, EMB), lambda i: (i, 0)),
            pl.BlockSpec((x.shape[1], H), lambda i: (0, 0)),
            pl.BlockSpec((1, H), lambda i: (0, 0)),
            pl.BlockSpec((EMB, H), lambda i: (0, 0)),
            pl.BlockSpec((H, H), lambda i: (0, 0)),
        ],
        out_specs=[
            pl.BlockSpec((_TN, H), lambda i: (i, 0)),
            pl.BlockSpec((_TN, H), lambda i: (i, 0)),
        ],
        out_shape=[
            jax.ShapeDtypeStruct((N, H), jnp.float32),
            jax.ShapeDtypeStruct((N, H), jnp.float32),
        ],
        compiler_params=pltpu.CompilerParams(
            dimension_semantics=(pltpu.PARALLEL,)),
    )(x, lv, wn, b, wl, wm)


def _node_mid_body(hpre_ref, pa_ref, pb_ref, lv_ref, wu_ref, wl_ref,
                   wm_ref, hpre1_ref, hm1_ref):
    agg = pa_ref[0] + pa_ref[1] + pb_ref[0] + pb_ref[1]
    h1 = jnp.maximum(
        hpre_ref[...]
        + jnp.dot(agg, wu_ref[...], preferred_element_type=jnp.float32), 0.0)
    hp1 = h1 + jnp.dot(lv_ref[...], wl_ref[...], preferred_element_type=jnp.float32)
    hpre1_ref[...] = hp1
    hm1_ref[...] = jnp.dot(hp1, wm_ref[...], preferred_element_type=jnp.float32)


def _node_mid(hpre, pa, pb, lv, wu, wl, wm):
    grid = (N // _TN,)
    pspec = pl.BlockSpec((NC, _TN, H), lambda i: (0, i, 0))
    return pl.pallas_call(
        _node_mid_body,
        grid=grid,
        in_specs=[
            pl.BlockSpec((_TN, H), lambda i: (i, 0)),
            pspec,
            pspec,
            pl.BlockSpec((_TN, EMB), lambda i: (i, 0)),
            pl.BlockSpec((H, H), lambda i: (0, 0)),
            pl.BlockSpec((EMB, H), lambda i: (0, 0)),
            pl.BlockSpec((H, H), lambda i: (0, 0)),
        ],
        out_specs=[
            pl.BlockSpec((_TN, H), lambda i: (i, 0)),
            pl.BlockSpec((_TN, H), lambda i: (i, 0)),
        ],
        out_shape=[
            jax.ShapeDtypeStruct((N, H), jnp.float32),
            jax.ShapeDtypeStruct((N, H), jnp.float32),
        ],
        compiler_params=pltpu.CompilerParams(
            dimension_semantics=(pltpu.PARALLEL,)),
    )(hpre, pa, pb, lv, wu, wl, wm)


def _node_out_body(hpre_ref, pa_ref, pb_ref, wu_ref, wh_ref, out_ref):
    agg = pa_ref[0] + pa_ref[1] + pb_ref[0] + pb_ref[1]
    h2 = jnp.maximum(
        hpre_ref[...]
        + jnp.dot(agg, wu_ref[...], preferred_element_type=jnp.float32), 0.0)
    out_ref[...] = jnp.dot(h2, wh_ref[...], preferred_element_type=jnp.float32)


def _node_out(hpre, pa, pb, wu, wh):
    grid = (N // _TN,)
    pspec = pl.BlockSpec((NC, _TN, H), lambda i: (0, i, 0))
    return pl.pallas_call(
        _node_out_body,
        grid=grid,
        in_specs=[
            pl.BlockSpec((_TN, H), lambda i: (i, 0)),
            pspec,
            pspec,
            pl.BlockSpec((H, H), lambda i: (0, 0)),
            pl.BlockSpec((H, wh.shape[1]), lambda i: (0, 0)),
        ],
        out_specs=pl.BlockSpec((_TN, wh.shape[1]), lambda i: (i, 0)),
        out_shape=jax.ShapeDtypeStruct((N, wh.shape[1]), jnp.float32),
        compiler_params=pltpu.CompilerParams(
            dimension_semantics=(pltpu.PARALLEL,)),
    )(hpre, pa, pb, wu, wh)


_sc_mesh = plsc.VectorSubcoreMesh(
    core_axis_name="c", subcore_axis_name="s", num_cores=NC, num_subcores=NS)


def _make_sc_edge(ew):
    nch = ew // C
    assert nch * C == ew and ew % 8 == 0 and nch >= 2

    @functools.partial(
        pl.kernel,
        out_type=jax.ShapeDtypeStruct((NC, NPAD, H), jnp.float32),
        mesh=_sc_mesh,
        compiler_params=pltpu.CompilerParams(needs_layout_passes=False),
        scratch_types=[
            pltpu.VMEM((2 * C,), jnp.int32),
            pltpu.VMEM((2 * C,), jnp.int32),
            pltpu.VMEM((C,), jnp.int32),
            pltpu.VMEM((C,), jnp.int32),
            pltpu.VMEM((C, H), jnp.float32),
            pltpu.VMEM((C, H), jnp.float32),
            pltpu.VMEM((C, H // 2), jnp.int32),
            pltpu.VMEM((C, H // 2), jnp.int32),
            pltpu.VMEM_SHARED((NPAD, H), jnp.float32),
            pltpu.SemaphoreType.DMA,
            pltpu.SemaphoreType.DMA,
            pltpu.SemaphoreType.DMA,
            pltpu.SemaphoreType.DMA,
            pltpu.SemaphoreType.DMA,
            pltpu.SemaphoreType.DMA,
            pltpu.SemaphoreType.DMA,
            pltpu.SemaphoreType.DMA,
        ],
    )
    def sc_edge(hm_hbm, g_hbm, idx_hbm, zero_hbm, out_hbm,
                ibuf0_v, ibuf1_v, dchunk0_v, dchunk1_v,
                rows0_v, rows1_v, gate0_v, gate1_v,
                acc_sh, semr0, semr1, seml0, seml1, semi0, semi1,
                sems0, sems1):
        c = lax.axis_index("c")
        s = lax.axis_index("s")
        wid = s * NC + c
        ibuf = (ibuf0_v, ibuf1_v)
        dchunk_v = (dchunk0_v, dchunk1_v)
        rows_v = (rows0_v, rows1_v)
        gate_v = (gate0_v, gate1_v)
        semr = (semr0, semr1)
        seml = (seml0, seml1)
        semi = (semi0, semi1)
        sems = (sems0, sems1)

        pltpu.sync_copy(zero_hbm.at[pl.ds(s * RPS, RPS)],
                        acc_sh.at[pl.ds(s * RPS, RPS)])
        plsc.subcore_barrier()

        def issue_idx(i, b):
            pltpu.async_copy(
                idx_hbm.at[pl.ds((wid * nch + i) * 2 * C, 2 * C)],
                ibuf[b], semi[b])

        def wait_idx(b):
            pltpu.make_async_copy(idx_hbm.at[pl.ds(0, 2 * C)], ibuf[b],
                                  semi[b]).wait()

        def issue_data(i, b):
            @pl.when(i >= 2)
            def _():
                pltpu.make_async_copy(hm_hbm.at[pl.ds(0, C)], rows_v[b],
                                      sems[b]).wait()
            pltpu.async_copy(hm_hbm.at[ibuf[b].at[pl.ds(0, C)]],
                             rows_v[b], semr[b])
            pltpu.async_copy(g_hbm.at[pl.ds(wid * ew + i * C, C)],
                             gate_v[b], seml[b])

        def drain_data(b):
            pltpu.make_async_copy(hm_hbm.at[pl.ds(0, C)], rows_v[b],
                                  semr[b]).wait()
            pltpu.make_async_copy(g_hbm.at[pl.ds(0, C)], gate_v[b],
                                  seml[b]).wait()

        def gate_and_scatter(b):
            @plsc.parallel_loop(0, C, 1, unroll=4)
            def mul_row(r):
                for k in range(H // 32):
                    gbf = plsc.bitcast(gate_v[b][r, pl.ds(k * 16, 16)],
                                       jnp.bfloat16)
                    ga, gb = plsc.unpack(
                        gbf, format=plsc.PackFormat.INTERLEAVED)
                    sla = pl.ds(k * 32, 16)
                    slb = pl.ds(k * 32 + 16, 16)
                    rows_v[b][r, sla] = rows_v[b][r, sla] * ga
                    rows_v[b][r, slb] = rows_v[b][r, slb] * gb
            for k in range(C // 16):
                dchunk_v[b][pl.ds(k * 16, 16)] = ibuf[b][pl.ds(C + k * 16, 16)]
            pltpu.async_copy(rows_v[b], acc_sh.at[dchunk_v[b]], sems[b],
                             add=True)

        issue_idx(0, 0)
        wait_idx(0)
        issue_data(0, 0)
        issue_idx(1, 1)

        def step(st, carry):
            for b in range(2):
                i = 2 * st + b

                @pl.when(i + 1 < nch)
                def _():
                    wait_idx(1 - b)
                    issue_data(i + 1, 1 - b)

                @pl.when(i < nch)
                def _():
                    drain_data(b)
                    gate_and_scatter(b)

                @pl.when(i + 2 < nch)
                def _():
                    issue_idx(i + 2, b)
            return carry

        lax.fori_loop(0, (nch + 1) // 2, step, 0)
        pltpu.make_async_copy(hm_hbm.at[pl.ds(0, C)], rows_v[1], sems[1]).wait()
        pltpu.make_async_copy(hm_hbm.at[pl.ds(0, C)], rows_v[0], sems[0]).wait()
        plsc.subcore_barrier()

        pltpu.sync_copy(acc_sh.at[pl.ds(s * RPS, RPS)],
                        out_hbm.at[c, pl.ds(s * RPS, RPS)])

    return sc_edge


_sc_edge_a = _make_sc_edge(EA // NW)
_sc_edge_b = _make_sc_edge(EB // NW)


_PEV = np.empty(H // 2, dtype=np.int32)
_POD = np.empty(H // 2, dtype=np.int32)
for _k in range(H // 32):
    for _j in range(16):
        _PEV[16 * _k + _j] = 32 * _k + _j
        _POD[16 * _k + _j] = 32 * _k + 16 + _j


def _idx2(src, dst):
    return jnp.stack([src.reshape(-1, C), dst.reshape(-1, C)],
                     axis=1).reshape(-1)


def kernel(x, edge_index, edge_attr, loop_val, poly_val,
           W_node, b_node, W_edge_enc, W_loop, W_conn, W_msg, W_upd, W_head):
    src = edge_index[0].astype(jnp.int32)
    dst = edge_index[1].astype(jnp.int32)
    ws = (W_edge_enc[:, _PEV], W_edge_enc[:, _POD],
          W_conn[0][:, _PEV], W_conn[0][:, _POD],
          W_conn[1][:, _PEV], W_conn[1][:, _POD])
    idx2a = _idx2(src[:EA], dst[:EA])
    idx2b = _idx2(src[EA:], dst[EA:])
    zeros = jnp.zeros((NPAD, H), jnp.float32)

    g1a, g2a = _gates_rng(edge_attr, poly_val, ws, 0, EA)
    hpre0, hm0 = _node_in(x, loop_val, W_node, b_node.reshape(1, H),
                          W_loop[0], W_msg[0])
    p0a = _sc_edge_a(hm0, g1a, idx2a, zeros)
    g1b, g2b = _gates_rng(edge_attr, poly_val, ws, EA // _TE, EB)
    p0b = _sc_edge_b(hm0, g1b, idx2b, zeros)
    hpre1, hm1 = _node_mid(hpre0, p0a, p0b, loop_val,
                           W_upd[0], W_loop[1], W_msg[1])
    p1a = _sc_edge_a(hm1, g2a, idx2a, zeros)
    p1b = _sc_edge_b(hm1, g2b, idx2b, zeros)
    out = _node_out(hpre1, p1a, p1b, W_upd[1], W_head)
    return out
